# baseline (speedup 1.0000x reference)
import jax
import jax.numpy as jnp
from jax import lax
from jax.experimental import pallas as pl
from jax.experimental.pallas import tpu as pltpu
from jax.experimental.pallas import tpu_sc as plsc

N = 10000
NP = 10240
E = 320000
D = 128
G = 16
NC = 2
NS = 16
NW = NC * NS
CHUNK = 128
NCH = 80
NBUF = 4
DH = D // 2
EPT = NCH * CHUNK
EPAD = NW * EPT
RPT = NP // NS

_MESH = plsc.VectorSubcoreMesh(core_axis_name="c", subcore_axis_name="s")


def _segsum_body(h_hbm, src_hbm, dst_hbm, zeros_hbm, out_hbm,
                 srcv, dstv, rows, accum, gsem):
    c = lax.axis_index("c")
    s = lax.axis_index("s")
    wid = c * NS + s
    base = s * RPT
    pltpu.sync_copy(src_hbm.at[wid], srcv)
    pltpu.sync_copy(dst_hbm.at[wid], dstv)
    pltpu.sync_copy(zeros_hbm.at[pl.ds(base, RPT)], accum.at[pl.ds(base, RPT)])
    plsc.subcore_barrier()

    def grp(j, carry):
        pltpu.async_copy(h_hbm.at[srcv.at[j]], rows, gsem).wait()
        pltpu.sync_copy(rows, accum.at[dstv.at[j]], add=True)
        return carry

    lax.fori_loop(0, NCH, grp, 0)
    plsc.subcore_barrier()
    pltpu.sync_copy(accum.at[pl.ds(base, RPT)], out_hbm.at[c, pl.ds(base, RPT)])


@jax.jit
def _segsum(h, src3, dst3, zeros_big):
    return pl.kernel(
        _segsum_body,
        out_type=jax.ShapeDtypeStruct((NC, NP, D), jnp.float32),
        mesh=_MESH,
        scratch_types=[
            pltpu.VMEM((NCH, CHUNK), jnp.int32),
            pltpu.VMEM((NCH, CHUNK), jnp.int32),
            pltpu.VMEM((CHUNK, D), jnp.float32),
            pltpu.VMEM_SHARED((NP, D), jnp.float32),
            pltpu.SemaphoreType.DMA,
        ],
    )(h, src3, dst3, zeros_big)


def _cnt_body(dst_hbm, zeros1_hbm, out_hbm, dstv, cnt):
    c = lax.axis_index("c")
    s = lax.axis_index("s")
    wid = c * NS + s
    pltpu.sync_copy(dst_hbm.at[pl.ds(wid * EPT, EPT)], dstv)
    pltpu.sync_copy(zeros1_hbm, cnt)
    ones = jnp.ones((16,), jnp.float32)

    def grp(j, carry):
        d = dstv[pl.ds(j * 16, 16)]
        plsc.addupdate_scatter(cnt, [d], ones)
        return carry

    lax.fori_loop(0, EPT // 16, grp, 0)
    pltpu.sync_copy(cnt, out_hbm.at[wid])


@jax.jit
def _cnt(dst1, zeros1):
    return pl.kernel(
        _cnt_body,
        out_type=jax.ShapeDtypeStruct((NW, NP), jnp.float32),
        mesh=_MESH,
        scratch_types=[
            pltpu.VMEM((EPT,), jnp.int32),
            pltpu.VMEM((NP,), jnp.float32),
        ],
        compiler_params=pltpu.CompilerParams(needs_layout_passes=False),
    )(dst1, zeros1)


_BN = 512


def _rcp_body(cnt_ref, rcp_ref):
    c = jnp.sum(cnt_ref[...], axis=0)
    rcp_ref[...] = 1.0 / jnp.maximum(c, 1.0)


@jax.jit
def _rcp(cntp):
    return pl.pallas_call(
        _rcp_body,
        out_shape=jax.ShapeDtypeStruct((NP, 1), jnp.float32),
        grid=(NP // _BN,),
        in_specs=[pl.BlockSpec((NW, _BN, 1), lambda i: (0, i, 0))],
        out_specs=pl.BlockSpec((_BN, 1), lambda i: (i, 0)),
    )(cntp)


def _layer_body(sp_ref, rcp_ref, h_ref, wl_ref, bl_ref, wr_ref, out_ref):
    agg = (sp_ref[0] + sp_ref[1]) * rcp_ref[...]
    dn = (((1,), (1,)), ((), ()))
    y = lax.dot_general(agg, wl_ref[...], dn, preferred_element_type=jnp.float32)
    y = y + bl_ref[...] + lax.dot_general(
        h_ref[...], wr_ref[...], dn, preferred_element_type=jnp.float32)
    out_ref[...] = jnp.maximum(y, 0.0)


@jax.jit
def _layer(sp, rcp, h, Wl, bl, Wr):
    return pl.pallas_call(
        _layer_body,
        out_shape=jax.ShapeDtypeStruct((NP, D), jnp.float32),
        grid=(NP // _BN,),
        in_specs=[
            pl.BlockSpec((NC, _BN, D), lambda i: (0, i, 0)),
            pl.BlockSpec((_BN, 1), lambda i: (i, 0)),
            pl.BlockSpec((_BN, D), lambda i: (i, 0)),
            pl.BlockSpec((D, D), lambda i: (0, 0)),
            pl.BlockSpec((1, D), lambda i: (0, 0)),
            pl.BlockSpec((D, D), lambda i: (0, 0)),
        ],
        out_specs=pl.BlockSpec((_BN, D), lambda i: (i, 0)),
    )(sp, rcp, h, Wl, bl, Wr)


def _pool_body(h_ref, b_ref, wc_ref, bc_ref, out_ref, acc, gacc):
    i = pl.program_id(0)

    @pl.when(i == 0)
    def _():
        acc[...] = jnp.zeros_like(acc)
        gacc[...] = jnp.zeros_like(gacc)

    onehot = (b_ref[...] == lax.broadcasted_iota(jnp.int32, (_BN, G), 1)
              ).astype(jnp.float32)
    dn0 = (((0,), (0,)), ((), ()))
    acc[...] += lax.dot_general(onehot, h_ref[...], dn0,
                                preferred_element_type=jnp.float32)
    gacc[...] += lax.dot_general(onehot, jnp.ones((_BN, 1), jnp.float32), dn0,
                                 preferred_element_type=jnp.float32)

    @pl.when(i == NP // _BN - 1)
    def _():
        pooled = acc[...] * (1.0 / jnp.maximum(gacc[...], 1.0))
        z = jnp.sum(pooled * wc_ref[...], axis=1, keepdims=True) + bc_ref[0, 0]
        out_ref[...] = 1.0 / (1.0 + jnp.exp(-z))


@jax.jit
def _pool(h, batch_p, Wc, bc):
    return pl.pallas_call(
        _pool_body,
        out_shape=jax.ShapeDtypeStruct((G, 1), jnp.float32),
        grid=(NP // _BN,),
        in_specs=[
            pl.BlockSpec((_BN, D), lambda i: (i, 0)),
            pl.BlockSpec((_BN, 1), lambda i: (i, 0)),
            pl.BlockSpec((1, D), lambda i: (0, 0)),
            pl.BlockSpec((1, 1), lambda i: (0, 0)),
        ],
        out_specs=pl.BlockSpec((G, 1), lambda i: (0, 0)),
        scratch_shapes=[
            pltpu.VMEM((G, D), jnp.float32),
            pltpu.VMEM((G, 1), jnp.float32),
        ],
    )(h, batch_p, Wc, bc)


def kernel(x, edge_index, edge_weight, batch,
           W1l, b1l, W1r, W2l, b2l, W2r, W3l, b3l, W3r, W4l, b4l, W4r, Wc, bc):
    src, dst = edge_index[0], edge_index[1]
    pad = EPAD - E
    src_p = jnp.concatenate([src, jnp.zeros((pad,), jnp.int32)])
    dst_p = jnp.concatenate([dst, jnp.full((pad,), N, jnp.int32)])
    src3 = src_p.reshape(NW, NCH, CHUNK)
    dst3 = dst_p.reshape(NW, NCH, CHUNK)
    xp = jnp.concatenate([x, jnp.zeros((NP - N, D), jnp.float32)])
    batch_p = jnp.concatenate(
        [batch, jnp.full((NP - N,), G, jnp.int32)]).reshape(NP, 1)
    zeros_big = jnp.zeros((NP, D), jnp.float32)
    zeros1 = jnp.zeros((NP,), jnp.float32)

    rcp = _rcp(_cnt(dst_p, zeros1).reshape(NW, NP, 1))

    h = xp
    for (Wl, bl, Wr) in ((W1l, b1l, W1r), (W2l, b2l, W2r),
                         (W3l, b3l, W3r), (W4l, b4l, W4r)):
        sp = _segsum(h, src3, dst3, zeros_big)
        h = _layer(sp, rcp, h, Wl, bl.reshape(1, D), Wr)

    return _pool(h, batch_p, Wc, bc.reshape(1, 1))


# speedup vs baseline: 1.4639x; 1.4639x over previous
import jax
import jax.numpy as jnp
from jax import lax
from jax.experimental import pallas as pl
from jax.experimental.pallas import tpu as pltpu
from jax.experimental.pallas import tpu_sc as plsc

N = 10000
NP = 10240
E = 320000
D = 128
G = 16
NC = 2
NS = 16
NW = NC * NS
CHUNK = 128
NCH = 79
EPT = NCH * CHUNK
EPAD = NW * EPT
RPT = NP // NS

_MESH = plsc.VectorSubcoreMesh(core_axis_name="c", subcore_axis_name="s")


def _segsum_body(h_hbm, src_hbm, dst_hbm, zeros_hbm, out_hbm,
                 srcv, dstv, rows, accum, sem):
    c = lax.axis_index("c")
    s = lax.axis_index("s")
    wid = c * NS + s
    pltpu.sync_copy(src_hbm.at[wid], srcv)
    pltpu.sync_copy(dst_hbm.at[wid], dstv)
    base = s * RPT
    pltpu.sync_copy(zeros_hbm.at[pl.ds(base, RPT)], accum.at[pl.ds(base, RPT)])
    plsc.subcore_barrier()

    def chunk(j, carry):
        pltpu.async_copy(h_hbm.at[srcv.at[j]], rows, sem).wait()
        pltpu.sync_copy(rows, accum.at[dstv.at[j]], add=True)
        return carry

    lax.fori_loop(0, NCH, chunk, 0)
    plsc.subcore_barrier()
    pltpu.sync_copy(accum.at[pl.ds(base, RPT)], out_hbm.at[c, pl.ds(base, RPT)])


@jax.jit
def _segsum(h, src3, dst3, zeros_big):
    return pl.kernel(
        _segsum_body,
        out_type=jax.ShapeDtypeStruct((NC, NP, D), jnp.float32),
        mesh=_MESH,
        scratch_types=[
            pltpu.VMEM((NCH, CHUNK), jnp.int32),
            pltpu.VMEM((NCH, CHUNK), jnp.int32),
            pltpu.VMEM((CHUNK, D), jnp.float32),
            pltpu.VMEM_SHARED((NP, D), jnp.float32),
            pltpu.SemaphoreType.DMA,
        ],
    )(h, src3, dst3, zeros_big)


def _cnt_body(dst_hbm, zeros1_hbm, out_hbm, dstv, cnt):
    c = lax.axis_index("c")
    s = lax.axis_index("s")
    wid = c * NS + s
    pltpu.sync_copy(dst_hbm.at[wid], dstv)
    pltpu.sync_copy(zeros1_hbm, cnt)
    ones = jnp.ones((16,), jnp.float32)

    def grp(j, carry):
        d = dstv[pl.ds(j * 16, 16)]
        plsc.addupdate_scatter(cnt, [d], ones)
        return carry

    lax.fori_loop(0, EPT // 16, grp, 0)
    pltpu.sync_copy(cnt, out_hbm.at[wid])


@jax.jit
def _cnt(dst2, zeros1):
    return pl.kernel(
        _cnt_body,
        out_type=jax.ShapeDtypeStruct((NW, NP), jnp.float32),
        mesh=_MESH,
        scratch_types=[
            pltpu.VMEM((EPT,), jnp.int32),
            pltpu.VMEM((NP,), jnp.float32),
        ],
        compiler_params=pltpu.CompilerParams(needs_layout_passes=False),
    )(dst2, zeros1)


_BN = 512


def _rcp_body(cnt_ref, rcp_ref):
    c = jnp.sum(cnt_ref[...], axis=0)
    rcp_ref[...] = 1.0 / jnp.maximum(c, 1.0)


@jax.jit
def _rcp(cntp):
    return pl.pallas_call(
        _rcp_body,
        out_shape=jax.ShapeDtypeStruct((NP, 1), jnp.float32),
        grid=(NP // _BN,),
        in_specs=[pl.BlockSpec((NW, _BN, 1), lambda i: (0, i, 0))],
        out_specs=pl.BlockSpec((_BN, 1), lambda i: (i, 0)),
    )(cntp)


def _layer_body(sp_ref, rcp_ref, h_ref, wl_ref, bl_ref, wr_ref, out_ref):
    agg = (sp_ref[0] + sp_ref[1]) * rcp_ref[...]
    dn = (((1,), (1,)), ((), ()))
    y = lax.dot_general(agg, wl_ref[...], dn, preferred_element_type=jnp.float32)
    y = y + bl_ref[...] + lax.dot_general(
        h_ref[...], wr_ref[...], dn, preferred_element_type=jnp.float32)
    out_ref[...] = jnp.maximum(y, 0.0)


@jax.jit
def _layer(sp, rcp, h, Wl, bl, Wr):
    return pl.pallas_call(
        _layer_body,
        out_shape=jax.ShapeDtypeStruct((NP, D), jnp.float32),
        grid=(NP // _BN,),
        in_specs=[
            pl.BlockSpec((NC, _BN, D), lambda i: (0, i, 0)),
            pl.BlockSpec((_BN, 1), lambda i: (i, 0)),
            pl.BlockSpec((_BN, D), lambda i: (i, 0)),
            pl.BlockSpec((D, D), lambda i: (0, 0)),
            pl.BlockSpec((1, D), lambda i: (0, 0)),
            pl.BlockSpec((D, D), lambda i: (0, 0)),
        ],
        out_specs=pl.BlockSpec((_BN, D), lambda i: (i, 0)),
    )(sp, rcp, h, Wl, bl, Wr)


def _pool_body(h_ref, b_ref, wc_ref, bc_ref, out_ref, acc, gacc):
    i = pl.program_id(0)

    @pl.when(i == 0)
    def _():
        acc[...] = jnp.zeros_like(acc)
        gacc[...] = jnp.zeros_like(gacc)

    onehot = (b_ref[...] == lax.broadcasted_iota(jnp.int32, (_BN, G), 1)
              ).astype(jnp.float32)
    dn0 = (((0,), (0,)), ((), ()))
    acc[...] += lax.dot_general(onehot, h_ref[...], dn0,
                                preferred_element_type=jnp.float32)
    gacc[...] += lax.dot_general(onehot, jnp.ones((_BN, 1), jnp.float32), dn0,
                                 preferred_element_type=jnp.float32)

    @pl.when(i == NP // _BN - 1)
    def _():
        pooled = acc[...] * (1.0 / jnp.maximum(gacc[...], 1.0))
        z = jnp.sum(pooled * wc_ref[...], axis=1, keepdims=True) + bc_ref[0, 0]
        out_ref[...] = 1.0 / (1.0 + jnp.exp(-z))


@jax.jit
def _pool(h, batch_p, Wc, bc):
    return pl.pallas_call(
        _pool_body,
        out_shape=jax.ShapeDtypeStruct((G, 1), jnp.float32),
        grid=(NP // _BN,),
        in_specs=[
            pl.BlockSpec((_BN, D), lambda i: (i, 0)),
            pl.BlockSpec((_BN, 1), lambda i: (i, 0)),
            pl.BlockSpec((1, D), lambda i: (0, 0)),
            pl.BlockSpec((1, 1), lambda i: (0, 0)),
        ],
        out_specs=pl.BlockSpec((G, 1), lambda i: (0, 0)),
        scratch_shapes=[
            pltpu.VMEM((G, D), jnp.float32),
            pltpu.VMEM((G, 1), jnp.float32),
        ],
    )(h, batch_p, Wc, bc)


def kernel(x, edge_index, edge_weight, batch,
           W1l, b1l, W1r, W2l, b2l, W2r, W3l, b3l, W3r, W4l, b4l, W4r, Wc, bc):
    src, dst = edge_index[0], edge_index[1]
    pad = EPAD - E
    src_p = jnp.concatenate([src, jnp.zeros((pad,), jnp.int32)])
    dst_p = jnp.concatenate([dst, jnp.full((pad,), N, jnp.int32)])
    src3 = src_p.reshape(NW, NCH, CHUNK)
    dst3 = dst_p.reshape(NW, NCH, CHUNK)
    dst2 = dst_p.reshape(NW, EPT)
    xp = jnp.concatenate([x, jnp.zeros((NP - N, D), jnp.float32)])
    batch_p = jnp.concatenate(
        [batch, jnp.full((NP - N,), G, jnp.int32)]).reshape(NP, 1)
    zeros_big = jnp.zeros((NP, D), jnp.float32)
    zeros1 = jnp.zeros((NP,), jnp.float32)

    rcp = _rcp(_cnt(dst2, zeros1).reshape(NW, NP, 1))

    h = xp
    for (Wl, bl, Wr) in ((W1l, b1l, W1r), (W2l, b2l, W2r),
                         (W3l, b3l, W3r), (W4l, b4l, W4r)):
        sp = _segsum(h, src3, dst3, zeros_big)
        h = _layer(sp, rcp, h, Wl, bl.reshape(1, D), Wr)

    return _pool(h, batch_p, Wc, bc.reshape(1, 1))
